# baseline (speedup 1.0000x reference)
import jax
import jax.numpy as jnp
from jax import lax
from jax.experimental import pallas as pl
from jax.experimental.pallas import tpu as pltpu
from jax.experimental.pallas import tpu_sc as plsc

F32 = jnp.float32
N_NODES = 10000
N_EDGES = 160000
NE_H = N_EDGES // 2
D = 128
E_BLK = 8000
GW = 128
N_SUBCORES = 16
NPAD = 10240
ROWS_PER_SUB = NPAD // N_SUBCORES

_LN_EPS = 1e-6


def _ln(x, g, b):
    mu = jnp.mean(x, axis=-1, keepdims=True)
    xc = x - mu
    var = jnp.mean(xc * xc, axis=-1, keepdims=True)
    return xc * lax.rsqrt(var + _LN_EPS) * g + b


def _mlp3(x, w1, b1, w2, b2, w3, b3):
    h = jnp.maximum(x @ w1[...] + b1[...], 0.0)
    h = jnp.maximum(h @ w2[...] + b2[...], 0.0)
    return h @ w3[...] + b3[...]


def _bdot(x, w):
    return jax.lax.dot(x.astype(jnp.bfloat16), w[...].astype(jnp.bfloat16),
                       preferred_element_type=F32)


def _mlp3b(x, w1, b1, w2, b2, w3, b3):
    h = jnp.maximum(_bdot(x, w1) + b1[...], 0.0)
    h = jnp.maximum(_bdot(h, w2) + b2[...], 0.0)
    return _bdot(h, w3) + b3[...]


def _flat(mp):
    (w1, b1), (w2, b2), (w3, b3) = mp["layers"]
    out = [w1, b1.reshape(1, -1), w2, b2.reshape(1, -1), w3, b3.reshape(1, -1)]
    if mp["ln"] is not None:
        out += [mp["ln"][0].reshape(1, -1), mp["ln"][1].reshape(1, -1)]
    return out


def _mlp_body(has_ln, *refs):
    if has_ln:
        x_ref, w1, b1, w2, b2, w3, b3, g, be, o_ref = refs
    else:
        x_ref, w1, b1, w2, b2, w3, b3, o_ref = refs
    h = _mlp3(x_ref[...], w1, b1, w2, b2, w3, b3)
    if has_ln:
        h = _ln(h, g[...], be[...])
    o_ref[...] = h


def _mlp_rows(x, mp):
    n = x.shape[0]
    dout = mp["layers"][2][0].shape[1]
    ln = mp["ln"]
    body = lambda *refs: _mlp_body(ln is not None, *refs)
    return pl.pallas_call(
        body, out_shape=jax.ShapeDtypeStruct((n, dout), F32),
    )(x, *_flat(mp))


def _enc_pre_body(nf, ctx, w1, b1, w2, b2, w3, b3, g, be,
                  wsrc, wdst, wec, be1, wnc, bn1,
                  nl_o, gsrc_o, gdst_o, bee_o, bne_o):
    nl = _ln(_mlp3b(nf[...], w1, b1, w2, b2, w3, b3), g[...], be[...])
    nl_o[...] = nl
    gsrc_o[...] = _bdot(nl, wsrc)
    gdst_o[...] = _bdot(nl, wdst)
    c = ctx[...]
    bee_o[...] = c @ wec[...] + be1[...]
    bne_o[...] = c @ wnc[...] + bn1[...]


def _enc_node_pre(node_features, ctx_lat, enc_mp, w1e, b1e, w1n, b1n):
    args = [node_features, ctx_lat] + _flat(enc_mp) + [
        w1e[128:256], w1e[256:384], w1e[384:512], b1e.reshape(1, -1),
        w1n[256:384], b1n.reshape(1, -1)]
    return pl.pallas_call(
        _enc_pre_body,
        out_shape=(jax.ShapeDtypeStruct((N_NODES, D), F32),
                   jax.ShapeDtypeStruct((N_NODES, D), F32),
                   jax.ShapeDtypeStruct((N_NODES, D), F32),
                   jax.ShapeDtypeStruct((1, D), F32),
                   jax.ShapeDtypeStruct((1, D), F32)),
    )(*args)


def _edge0_body(efp, gsd, ebd, eb1, ew2, eb2, ew3, eb3, eg, ebe,
                w1, bee, w2, b2, w3, b3, g, be, ne_o, el_o):
    y = _bdot(efp[...], ebd).reshape(E_BLK, D)
    hh = jnp.maximum(y + eb1[...], 0.0)
    hh = jnp.maximum(_bdot(hh, ew2) + eb2[...], 0.0)
    hh = _bdot(hh, ew3) + eb3[...]
    el = _ln(hh, eg[...], ebe[...])
    h = jnp.maximum(_bdot(el, w1) + gsd[...] + bee[...], 0.0)
    h = jnp.maximum(_bdot(h, w2) + b2[...], 0.0)
    h = _bdot(h, w3) + b3[...]
    ne = _ln(h, g[...], be[...])
    ne_o[...] = ne
    el_o[...] = (el + ne).astype(jnp.bfloat16)


def _edge_step0(h, efp, ebd, gsd, enc_mp, w1e, bee, mp):
    n = gsd.shape[0]
    nblk = n // E_BLK
    off = h * nblk
    args = ([efp, gsd, ebd] + _flat(enc_mp)[1:]
            + [w1e[0:128], bee] + _flat(mp)[2:])
    blk = pl.BlockSpec((E_BLK, D), lambda i: (i, 0))
    blkp = pl.BlockSpec((E_BLK // 8, D), lambda i: (i + off, 0))
    full = lambda a: pl.BlockSpec(a.shape, lambda i: (0,) * a.ndim)
    return pl.pallas_call(
        _edge0_body,
        grid=(nblk,),
        in_specs=[blkp, blk] + [full(a) for a in args[2:]],
        out_specs=[blk, blk],
        out_shape=(jax.ShapeDtypeStruct((n, D), F32),
                   jax.ShapeDtypeStruct((n, D), jnp.bfloat16)),
    )(*args)


def _edge1_body(*refs):
    (el_ref, gsd, w1, bee, w2, b2, w3, b3, g, be,
     dw1, db1, dw2, db2, dw3, db3) = refs[:16]
    ne_o, eo_o = refs[-2], refs[-1]
    el = el_ref[...]
    h = jnp.maximum(jax.lax.dot(el, w1[...].astype(jnp.bfloat16),
                                preferred_element_type=F32)
                    + gsd[...] + bee[...], 0.0)
    h = jnp.maximum(_bdot(h, w2) + b2[...], 0.0)
    h = _bdot(h, w3) + b3[...]
    ne = _ln(h, g[...], be[...])
    ne_o[...] = ne
    eo_o[...] = _mlp3b(el.astype(F32) + ne, dw1, db1, dw2, db2, dw3, db3)


def _edge_step1(h, edge_lat, gsd, w1e, bee, mp, dec_mp, eo_prev=None):
    n = edge_lat.shape[0]
    nblk = n // E_BLK
    off = h * nblk
    args = ([edge_lat, gsd, w1e[0:128], bee]
            + _flat(mp)[2:] + _flat(dec_mp))
    blk = pl.BlockSpec((E_BLK, D), lambda i: (i, 0))
    oblk = pl.BlockSpec((E_BLK, D), lambda i: (i + off, 0))
    full = lambda a: pl.BlockSpec(a.shape, lambda i: (0,) * a.ndim)
    in_specs = [blk, blk] + [full(a) for a in args[2:]]
    aliases = {}
    if eo_prev is not None:
        aliases = {len(args): 1}
        args = args + [eo_prev]
        in_specs = in_specs + [pl.BlockSpec(memory_space=pl.ANY)]
    return pl.pallas_call(
        _edge1_body,
        grid=(nblk,),
        in_specs=in_specs,
        out_specs=[blk, oblk],
        out_shape=(jax.ShapeDtypeStruct((n, D), F32),
                   jax.ShapeDtypeStruct((N_EDGES, D), F32)),
        input_output_aliases=aliases,
    )(*args)


_N_CORE = 23


def _node_core(nl_ref, p2a_ref, p2b_ref, ctx_ref, wnl, wnp, bneff, wn2, bn2,
               wn3, bn3, gn, ben, wcc, wcn, wce, bc1, wc2, bc2, wc3, bc3,
               gc, bec):
    nl = nl_ref[...]
    pooled = (p2a_ref[0] + p2a_ref[1] + p2b_ref[0] + p2b_ref[1])[:N_NODES]
    h = jnp.maximum(_bdot(nl, wnl) + _bdot(pooled, wnp) + bneff[...], 0.0)
    h = jnp.maximum(_bdot(h, wn2) + bn2[...], 0.0)
    h = _bdot(h, wn3) + bn3[...]
    nn = _ln(h, gn[...], ben[...])
    snn = jnp.sum(nn, axis=0, keepdims=True)
    sne = jnp.sum(pooled, axis=0, keepdims=True)
    c = ctx_ref[...]
    hc = jnp.maximum(c @ wcc[...] + snn @ wcn[...] + sne @ wce[...]
                     + bc1[...], 0.0)
    hc = jnp.maximum(hc @ wc2[...] + bc2[...], 0.0)
    hc = hc @ wc3[...] + bc3[...]
    return nl + nn, c + _ln(hc, gc[...], bec[...])


def _node_pre_body(*refs):
    (core, (wsrc, wdst, wec, be1, wnc, bn1),
     (nlo, ctxo, gsrc_o, gdst_o, bee_o, bne_o)) = (
        refs[:_N_CORE], refs[_N_CORE:_N_CORE + 6], refs[_N_CORE + 6:])
    nl_new, ctx_new = _node_core(*core)
    nlo[...] = nl_new
    ctxo[...] = ctx_new
    gsrc_o[...] = _bdot(nl_new, wsrc)
    gdst_o[...] = _bdot(nl_new, wdst)
    bee_o[...] = ctx_new @ wec[...] + be1[...]
    bne_o[...] = ctx_new @ wnc[...] + bn1[...]


def _node_dec_body(*refs):
    (core, (ndw1, ndb1, ndw2, ndb2, ndw3, ndb3,
            cdw1, cdb1, cdw2, cdb2, cdw3, cdb3),
     (no_o, co_o)) = (
        refs[:_N_CORE], refs[_N_CORE:_N_CORE + 12], refs[_N_CORE + 12:])
    nl_new, ctx_new = _node_core(*core)
    no_o[...] = _mlp3b(nl_new, ndw1, ndb1, ndw2, ndb2, ndw3, ndb3)
    co_o[...] = _mlp3(ctx_new, cdw1, cdb1, cdw2, cdb2, cdw3, cdb3)


def _node_core_args(node_lat, p2a, p2b, ctx_lat, w1n, bne, node_mp, ctx_mp):
    wc1 = ctx_mp["layers"][0][0]
    bc1 = ctx_mp["layers"][0][1]
    return ([node_lat, p2a, p2b, ctx_lat, w1n[0:128], w1n[128:256], bne]
            + _flat(node_mp)[2:]
            + [wc1[0:128], wc1[128:256], wc1[256:384], bc1.reshape(1, -1)]
            + _flat(ctx_mp)[2:])


def _node_step_pre(node_lat, p2a, p2b, ctx_lat, w1n, bne, node_mp, ctx_mp,
                   w1e_n, b1e_n, w1n_n, b1n_n):
    args = _node_core_args(node_lat, p2a, p2b, ctx_lat, w1n, bne,
                           node_mp, ctx_mp) + [
        w1e_n[128:256], w1e_n[256:384], w1e_n[384:512],
        b1e_n.reshape(1, -1), w1n_n[256:384], b1n_n.reshape(1, -1)]
    return pl.pallas_call(
        _node_pre_body,
        out_shape=(jax.ShapeDtypeStruct((N_NODES, D), F32),
                   jax.ShapeDtypeStruct((1, D), F32),
                   jax.ShapeDtypeStruct((N_NODES, D), F32),
                   jax.ShapeDtypeStruct((N_NODES, D), F32),
                   jax.ShapeDtypeStruct((1, D), F32),
                   jax.ShapeDtypeStruct((1, D), F32)),
    )(*args)


def _node_step_dec(node_lat, p2a, p2b, ctx_lat, w1n, bne, node_mp, ctx_mp,
                   dec_node_mp, dec_ctx_mp):
    args = (_node_core_args(node_lat, p2a, p2b, ctx_lat, w1n, bne,
                            node_mp, ctx_mp)
            + _flat(dec_node_mp) + _flat(dec_ctx_mp))
    return pl.pallas_call(
        _node_dec_body,
        out_shape=(jax.ShapeDtypeStruct((N_NODES, D), F32),
                   jax.ShapeDtypeStruct((1, D), F32)),
    )(*args)


_SC_MESH = plsc.VectorSubcoreMesh(core_axis_name="core",
                                  subcore_axis_name="subcore")


def _sc_gather(gsrc, gdst, src_idx, dst_idx):
    n = src_idx.shape[1]
    nwin = n // GW
    npw = (nwin + 31) // 32
    span = npw * GW

    @pl.kernel(out_type=jax.ShapeDtypeStruct((n, D), F32),
               mesh=_SC_MESH,
               scratch_types=[
                   pltpu.VMEM((span,), jnp.int32),
                   pltpu.VMEM((span,), jnp.int32),
                   pltpu.VMEM((2, GW, D), F32),
                   pltpu.VMEM((2, GW, D), F32),
                   pltpu.SemaphoreType.DMA, pltpu.SemaphoreType.DMA,
                   pltpu.SemaphoreType.DMA, pltpu.SemaphoreType.DMA,
                   pltpu.SemaphoreType.DMA, pltpu.SemaphoreType.DMA])
    def k(gsrc_hbm, gdst_hbm, si_hbm, di_hbm, o_hbm, idx_s, idx_d,
          acc, tmp, ss0, ss1, sd0, sd1, so0, so1):
        w = lax.axis_index("subcore") * 2 + lax.axis_index("core")
        nj = jnp.clip(nwin - w * npw, 0, npw)
        row0 = w * span
        sem_s, sem_d, sem_o = (ss0, ss1), (sd0, sd1), (so0, so1)

        @pl.when(nj > 0)
        def _():
            pltpu.sync_copy(si_hbm.at[0, pl.ds(row0, span)], idx_s)
            pltpu.sync_copy(di_hbm.at[0, pl.ds(row0, span)], idx_d)

        def issue(kk, b):
            @pl.when(kk >= 2)
            def _():
                pltpu.make_async_copy(acc.at[b], o_hbm.at[pl.ds(row0, GW)],
                                      sem_o[b]).wait()
            isl = idx_s.at[pl.ds(kk * GW, GW)]
            idl = idx_d.at[pl.ds(kk * GW, GW)]
            pltpu.async_copy(gsrc_hbm.at[isl], acc.at[b], sem_s[b])
            pltpu.async_copy(gdst_hbm.at[idl], tmp.at[b], sem_d[b])

        @pl.when(nj > 0)
        def _():
            issue(0, 0)

        @pl.loop(0, npw, step=2)
        def _(j):
            for b in range(2):
                kk = j + b
                nb = 1 - b

                @pl.when(kk + 1 < nj)
                def _():
                    issue(kk + 1, nb)

                @pl.when(kk < nj)
                def _():
                    pltpu.make_async_copy(gsrc_hbm.at[idx_s.at[pl.ds(0, GW)]],
                                          acc.at[b], sem_s[b]).wait()
                    pltpu.make_async_copy(gdst_hbm.at[idx_d.at[pl.ds(0, GW)]],
                                          tmp.at[b], sem_d[b]).wait()

                    @pl.loop(0, GW)
                    def _(r):
                        for c in range(8):
                            sl = pl.ds(c * 16, 16)
                            acc[b, r, sl] = acc[b, r, sl] + tmp[b, r, sl]

                    pltpu.async_copy(
                        acc.at[b], o_hbm.at[pl.ds(row0 + kk * GW, GW)],
                        sem_o[b])

        @pl.when(nj >= 2)
        def _():
            pltpu.make_async_copy(acc.at[0], o_hbm.at[pl.ds(row0, GW)],
                                  so0).wait()
            pltpu.make_async_copy(acc.at[1], o_hbm.at[pl.ds(row0, GW)],
                                  so1).wait()

    pad = 32 * span - n
    if pad:
        src_idx = jnp.pad(src_idx, ((0, 0), (0, pad)))
        dst_idx = jnp.pad(dst_idx, ((0, 0), (0, pad)))
    return k(gsrc, gdst, src_idx, dst_idx)


def _sc_segment_sum(new_edge, dst_idx, zeros):
    n = dst_idx.shape[1]

    @pl.kernel(out_type=jax.ShapeDtypeStruct((2, NPAD, D), F32),
               mesh=_SC_MESH,
               scratch_types=[pltpu.VMEM_SHARED((NPAD, D), F32)])
    def k(ne_hbm, di_hbm, z_hbm, o_hbm, acc):
        cid = lax.axis_index("core")
        sid = lax.axis_index("subcore")
        rows = pl.ds(sid * ROWS_PER_SUB, ROWS_PER_SUB)
        pltpu.sync_copy(z_hbm.at[rows], acc.at[rows])
        plsc.subcore_barrier()

        def body(di_vmem, ne_vmem):
            pltpu.sync_copy(ne_vmem, acc.at[di_vmem.at[0]], add=True)

        pltpu.emit_pipeline(
            body,
            grid=(n // GW,),
            in_specs=[pl.BlockSpec((1, GW), lambda i: (0, i)),
                      pl.BlockSpec((GW, D), lambda i: (i, 0))],
            out_specs=[],
            core_axis_name=("core", "subcore"),
            dimension_semantics=(pltpu.PARALLEL,),
        )(di_hbm, ne_hbm)
        plsc.subcore_barrier()
        pltpu.sync_copy(acc.at[rows], o_hbm.at[cid, rows])

    return k(new_edge, dst_idx, zeros)


def kernel(node_features, edge_features, context_features, edge_index,
           params):
    p = params
    src = [edge_index[0, :NE_H].reshape(1, NE_H),
           edge_index[0, NE_H:].reshape(1, NE_H)]
    dst = [edge_index[1, :NE_H].reshape(1, NE_H),
           edge_index[1, NE_H:].reshape(1, NE_H)]
    zeros = jnp.zeros((NPAD, D), F32)

    w1e = [p["proc"][s]["edge"]["layers"][0][0] for s in range(2)]
    b1e = [p["proc"][s]["edge"]["layers"][0][1] for s in range(2)]
    w1n = [p["proc"][s]["node"]["layers"][0][0] for s in range(2)]
    b1n = [p["proc"][s]["node"]["layers"][0][1] for s in range(2)]

    ctx_lat = _mlp_rows(context_features, p["enc_ctx"])
    node_lat, gsrc, gdst, bee, bne = _enc_node_pre(
        node_features, ctx_lat, p["enc_node"], w1e[0], b1e[0],
        w1n[0], b1n[0])

    el, p2 = [None, None], [None, None]
    g0 = _sc_gather(gsrc, gdst, src[0], dst[0])
    g1 = _sc_gather(gsrc, gdst, src[1], dst[1])
    efp = edge_features.reshape(N_EDGES // 8, 128)
    ebd = jax.scipy.linalg.block_diag(
        *([p["enc_edge"]["layers"][0][0]] * 8))
    for h in range(2):
        ne, el[h] = _edge_step0(h, efp, ebd, (g0, g1)[h],
                                p["enc_edge"], w1e[0], bee,
                                p["proc"][0]["edge"])
        p2[h] = _sc_segment_sum(ne, dst[h], zeros)
    (node_lat, ctx_lat, gsrc, gdst, bee, bne) = _node_step_pre(
        node_lat, p2[0], p2[1], ctx_lat, w1n[0], bne,
        p["proc"][0]["node"], p["proc"][0]["ctx"],
        w1e[1], b1e[1], w1n[1], b1n[1])

    edge_out = None
    g0 = _sc_gather(gsrc, gdst, src[0], dst[0])
    g1 = _sc_gather(gsrc, gdst, src[1], dst[1])
    for h in range(2):
        ne, edge_out = _edge_step1(h, el[h], (g0, g1)[h], w1e[1], bee,
                                   p["proc"][1]["edge"], p["dec_edge"],
                                   eo_prev=edge_out)
        p2[h] = _sc_segment_sum(ne, dst[h], zeros)
    node_out, ctx_out = _node_step_dec(
        node_lat, p2[0], p2[1], ctx_lat, w1n[1], bne,
        p["proc"][1]["node"], p["proc"][1]["ctx"],
        p["dec_node"], p["dec_ctx"])

    return (node_out, edge_out, ctx_out)


# speedup vs baseline: 1.0130x; 1.0130x over previous
import jax
import jax.numpy as jnp
from jax import lax
from jax.experimental import pallas as pl
from jax.experimental.pallas import tpu as pltpu
from jax.experimental.pallas import tpu_sc as plsc

F32 = jnp.float32
N_NODES = 10000
N_EDGES = 160000
NE_H = N_EDGES // 2
D = 128
E_BLK = 8000
GW = 128
N_SUBCORES = 16
NPAD = 10240
ROWS_PER_SUB = NPAD // N_SUBCORES

_LN_EPS = 1e-6


def _ln(x, g, b):
    mu = jnp.mean(x, axis=-1, keepdims=True)
    xc = x - mu
    var = jnp.mean(xc * xc, axis=-1, keepdims=True)
    return xc * lax.rsqrt(var + _LN_EPS) * g + b


def _mlp3(x, w1, b1, w2, b2, w3, b3):
    h = jnp.maximum(x @ w1[...] + b1[...], 0.0)
    h = jnp.maximum(h @ w2[...] + b2[...], 0.0)
    return h @ w3[...] + b3[...]


def _bdot(x, w):
    return jax.lax.dot(x.astype(jnp.bfloat16), w[...].astype(jnp.bfloat16),
                       preferred_element_type=F32)


def _mlp3b(x, w1, b1, w2, b2, w3, b3):
    h = jnp.maximum(_bdot(x, w1) + b1[...], 0.0)
    h = jnp.maximum(_bdot(h, w2) + b2[...], 0.0)
    return _bdot(h, w3) + b3[...]


def _flat(mp):
    (w1, b1), (w2, b2), (w3, b3) = mp["layers"]
    out = [w1, b1.reshape(1, -1), w2, b2.reshape(1, -1), w3, b3.reshape(1, -1)]
    if mp["ln"] is not None:
        out += [mp["ln"][0].reshape(1, -1), mp["ln"][1].reshape(1, -1)]
    return out


def _mlp_body(has_ln, *refs):
    if has_ln:
        x_ref, w1, b1, w2, b2, w3, b3, g, be, o_ref = refs
    else:
        x_ref, w1, b1, w2, b2, w3, b3, o_ref = refs
    h = _mlp3(x_ref[...], w1, b1, w2, b2, w3, b3)
    if has_ln:
        h = _ln(h, g[...], be[...])
    o_ref[...] = h


def _mlp_rows(x, mp):
    n = x.shape[0]
    dout = mp["layers"][2][0].shape[1]
    ln = mp["ln"]
    body = lambda *refs: _mlp_body(ln is not None, *refs)
    return pl.pallas_call(
        body, out_shape=jax.ShapeDtypeStruct((n, dout), F32),
    )(x, *_flat(mp))


def _enc_pre_body(nf, ctx, w1, b1, w2, b2, w3, b3, g, be,
                  wsrc, wdst, wec, be1, wnc, bn1,
                  nl_o, gsrc_o, gdst_o, bee_o, bne_o):
    nl = _ln(_mlp3(nf[...], w1, b1, w2, b2, w3, b3), g[...], be[...])
    nl_o[...] = nl
    gsrc_o[...] = nl @ wsrc[...]
    gdst_o[...] = nl @ wdst[...]
    c = ctx[...]
    bee_o[...] = c @ wec[...] + be1[...]
    bne_o[...] = c @ wnc[...] + bn1[...]


def _enc_node_pre(node_features, ctx_lat, enc_mp, w1e, b1e, w1n, b1n):
    args = [node_features, ctx_lat] + _flat(enc_mp) + [
        w1e[128:256], w1e[256:384], w1e[384:512], b1e.reshape(1, -1),
        w1n[256:384], b1n.reshape(1, -1)]
    return pl.pallas_call(
        _enc_pre_body,
        out_shape=(jax.ShapeDtypeStruct((N_NODES, D), F32),
                   jax.ShapeDtypeStruct((N_NODES, D), F32),
                   jax.ShapeDtypeStruct((N_NODES, D), F32),
                   jax.ShapeDtypeStruct((1, D), F32),
                   jax.ShapeDtypeStruct((1, D), F32)),
    )(*args)


def _edge0_body(efp, gsd, ebd, eb1, ew2, eb2, ew3, eb3, eg, ebe,
                w1, bee, w2, b2, w3, b3, g, be, ne_o, el_o):
    y = _bdot(efp[...], ebd).reshape(E_BLK, D)
    hh = jnp.maximum(y + eb1[...], 0.0)
    hh = jnp.maximum(_bdot(hh, ew2) + eb2[...], 0.0)
    hh = _bdot(hh, ew3) + eb3[...]
    el = _ln(hh, eg[...], ebe[...])
    h = jnp.maximum(_bdot(el, w1) + gsd[...] + bee[...], 0.0)
    h = jnp.maximum(_bdot(h, w2) + b2[...], 0.0)
    h = _bdot(h, w3) + b3[...]
    ne = _ln(h, g[...], be[...])
    ne_o[...] = ne
    el_o[...] = (el + ne).astype(jnp.bfloat16)


def _edge_step0(h, efp, ebd, gsd, enc_mp, w1e, bee, mp):
    n = gsd.shape[0]
    nblk = n // E_BLK
    off = h * nblk
    args = ([efp, gsd, ebd] + _flat(enc_mp)[1:]
            + [w1e[0:128], bee] + _flat(mp)[2:])
    blk = pl.BlockSpec((E_BLK, D), lambda i: (i, 0))
    blkp = pl.BlockSpec((E_BLK // 8, D), lambda i: (i + off, 0))
    full = lambda a: pl.BlockSpec(a.shape, lambda i: (0,) * a.ndim)
    return pl.pallas_call(
        _edge0_body,
        grid=(nblk,),
        in_specs=[blkp, blk] + [full(a) for a in args[2:]],
        out_specs=[blk, blk],
        out_shape=(jax.ShapeDtypeStruct((n, D), F32),
                   jax.ShapeDtypeStruct((n, D), jnp.bfloat16)),
    )(*args)


def _edge1_body(*refs):
    (el_ref, gsd, w1, bee, w2, b2, w3, b3, g, be,
     dw1, db1, dw2, db2, dw3, db3) = refs[:16]
    ne_o, eo_o = refs[-2], refs[-1]
    el = el_ref[...]
    h = jnp.maximum(jax.lax.dot(el, w1[...].astype(jnp.bfloat16),
                                preferred_element_type=F32)
                    + gsd[...] + bee[...], 0.0)
    h = jnp.maximum(_bdot(h, w2) + b2[...], 0.0)
    h = _bdot(h, w3) + b3[...]
    ne = _ln(h, g[...], be[...])
    ne_o[...] = ne
    eo_o[...] = _mlp3b(el.astype(F32) + ne, dw1, db1, dw2, db2, dw3, db3)


def _edge_step1(h, edge_lat, gsd, w1e, bee, mp, dec_mp, eo_prev=None):
    n = edge_lat.shape[0]
    nblk = n // E_BLK
    off = h * nblk
    args = ([edge_lat, gsd, w1e[0:128], bee]
            + _flat(mp)[2:] + _flat(dec_mp))
    blk = pl.BlockSpec((E_BLK, D), lambda i: (i, 0))
    oblk = pl.BlockSpec((E_BLK, D), lambda i: (i + off, 0))
    full = lambda a: pl.BlockSpec(a.shape, lambda i: (0,) * a.ndim)
    in_specs = [blk, blk] + [full(a) for a in args[2:]]
    aliases = {}
    if eo_prev is not None:
        aliases = {len(args): 1}
        args = args + [eo_prev]
        in_specs = in_specs + [pl.BlockSpec(memory_space=pl.ANY)]
    return pl.pallas_call(
        _edge1_body,
        grid=(nblk,),
        in_specs=in_specs,
        out_specs=[blk, oblk],
        out_shape=(jax.ShapeDtypeStruct((n, D), F32),
                   jax.ShapeDtypeStruct((N_EDGES, D), F32)),
        input_output_aliases=aliases,
    )(*args)


_N_CORE = 23


def _node_core(nl_ref, p2a_ref, p2b_ref, ctx_ref, wnl, wnp, bneff, wn2, bn2,
               wn3, bn3, gn, ben, wcc, wcn, wce, bc1, wc2, bc2, wc3, bc3,
               gc, bec):
    nl = nl_ref[...]
    pooled = (p2a_ref[0] + p2a_ref[1] + p2b_ref[0] + p2b_ref[1])[:N_NODES]
    h = jnp.maximum(nl @ wnl[...] + pooled @ wnp[...] + bneff[...], 0.0)
    h = jnp.maximum(h @ wn2[...] + bn2[...], 0.0)
    h = h @ wn3[...] + bn3[...]
    nn = _ln(h, gn[...], ben[...])
    snn = jnp.sum(nn, axis=0, keepdims=True)
    sne = jnp.sum(pooled, axis=0, keepdims=True)
    c = ctx_ref[...]
    hc = jnp.maximum(c @ wcc[...] + snn @ wcn[...] + sne @ wce[...]
                     + bc1[...], 0.0)
    hc = jnp.maximum(hc @ wc2[...] + bc2[...], 0.0)
    hc = hc @ wc3[...] + bc3[...]
    return nl + nn, c + _ln(hc, gc[...], bec[...])


def _node_pre_body(*refs):
    (core, (wsrc, wdst, wec, be1, wnc, bn1),
     (nlo, ctxo, gsrc_o, gdst_o, bee_o, bne_o)) = (
        refs[:_N_CORE], refs[_N_CORE:_N_CORE + 6], refs[_N_CORE + 6:])
    nl_new, ctx_new = _node_core(*core)
    nlo[...] = nl_new
    ctxo[...] = ctx_new
    gsrc_o[...] = nl_new @ wsrc[...]
    gdst_o[...] = nl_new @ wdst[...]
    bee_o[...] = ctx_new @ wec[...] + be1[...]
    bne_o[...] = ctx_new @ wnc[...] + bn1[...]


def _node_dec_body(*refs):
    (core, (ndw1, ndb1, ndw2, ndb2, ndw3, ndb3,
            cdw1, cdb1, cdw2, cdb2, cdw3, cdb3),
     (no_o, co_o)) = (
        refs[:_N_CORE], refs[_N_CORE:_N_CORE + 12], refs[_N_CORE + 12:])
    nl_new, ctx_new = _node_core(*core)
    no_o[...] = _mlp3(nl_new, ndw1, ndb1, ndw2, ndb2, ndw3, ndb3)
    co_o[...] = _mlp3(ctx_new, cdw1, cdb1, cdw2, cdb2, cdw3, cdb3)


def _node_core_args(node_lat, p2a, p2b, ctx_lat, w1n, bne, node_mp, ctx_mp):
    wc1 = ctx_mp["layers"][0][0]
    bc1 = ctx_mp["layers"][0][1]
    return ([node_lat, p2a, p2b, ctx_lat, w1n[0:128], w1n[128:256], bne]
            + _flat(node_mp)[2:]
            + [wc1[0:128], wc1[128:256], wc1[256:384], bc1.reshape(1, -1)]
            + _flat(ctx_mp)[2:])


def _node_step_pre(node_lat, p2a, p2b, ctx_lat, w1n, bne, node_mp, ctx_mp,
                   w1e_n, b1e_n, w1n_n, b1n_n):
    args = _node_core_args(node_lat, p2a, p2b, ctx_lat, w1n, bne,
                           node_mp, ctx_mp) + [
        w1e_n[128:256], w1e_n[256:384], w1e_n[384:512],
        b1e_n.reshape(1, -1), w1n_n[256:384], b1n_n.reshape(1, -1)]
    return pl.pallas_call(
        _node_pre_body,
        out_shape=(jax.ShapeDtypeStruct((N_NODES, D), F32),
                   jax.ShapeDtypeStruct((1, D), F32),
                   jax.ShapeDtypeStruct((N_NODES, D), F32),
                   jax.ShapeDtypeStruct((N_NODES, D), F32),
                   jax.ShapeDtypeStruct((1, D), F32),
                   jax.ShapeDtypeStruct((1, D), F32)),
    )(*args)


def _node_step_dec(node_lat, p2a, p2b, ctx_lat, w1n, bne, node_mp, ctx_mp,
                   dec_node_mp, dec_ctx_mp):
    args = (_node_core_args(node_lat, p2a, p2b, ctx_lat, w1n, bne,
                            node_mp, ctx_mp)
            + _flat(dec_node_mp) + _flat(dec_ctx_mp))
    return pl.pallas_call(
        _node_dec_body,
        out_shape=(jax.ShapeDtypeStruct((N_NODES, D), F32),
                   jax.ShapeDtypeStruct((1, D), F32)),
    )(*args)


_SC_MESH = plsc.VectorSubcoreMesh(core_axis_name="core",
                                  subcore_axis_name="subcore")


def _sc_gather(gsrc, gdst, src_idx, dst_idx):
    n = src_idx.shape[1]
    nwin = n // GW
    npw = (nwin + 31) // 32
    span = npw * GW

    @pl.kernel(out_type=jax.ShapeDtypeStruct((n, D), F32),
               mesh=_SC_MESH,
               scratch_types=[
                   pltpu.VMEM((span,), jnp.int32),
                   pltpu.VMEM((span,), jnp.int32),
                   pltpu.VMEM((2, GW, D), F32),
                   pltpu.VMEM((2, GW, D), F32),
                   pltpu.SemaphoreType.DMA, pltpu.SemaphoreType.DMA,
                   pltpu.SemaphoreType.DMA, pltpu.SemaphoreType.DMA,
                   pltpu.SemaphoreType.DMA, pltpu.SemaphoreType.DMA])
    def k(gsrc_hbm, gdst_hbm, si_hbm, di_hbm, o_hbm, idx_s, idx_d,
          acc, tmp, ss0, ss1, sd0, sd1, so0, so1):
        w = lax.axis_index("subcore") * 2 + lax.axis_index("core")
        nj = jnp.clip(nwin - w * npw, 0, npw)
        row0 = w * span
        sem_s, sem_d, sem_o = (ss0, ss1), (sd0, sd1), (so0, so1)

        @pl.when(nj > 0)
        def _():
            pltpu.sync_copy(si_hbm.at[0, pl.ds(row0, span)], idx_s)
            pltpu.sync_copy(di_hbm.at[0, pl.ds(row0, span)], idx_d)

        def issue(kk, b):
            @pl.when(kk >= 2)
            def _():
                pltpu.make_async_copy(acc.at[b], o_hbm.at[pl.ds(row0, GW)],
                                      sem_o[b]).wait()
            isl = idx_s.at[pl.ds(kk * GW, GW)]
            idl = idx_d.at[pl.ds(kk * GW, GW)]
            pltpu.async_copy(gsrc_hbm.at[isl], acc.at[b], sem_s[b])
            pltpu.async_copy(gdst_hbm.at[idl], tmp.at[b], sem_d[b])

        @pl.when(nj > 0)
        def _():
            issue(0, 0)

        @pl.loop(0, npw, step=2)
        def _(j):
            for b in range(2):
                kk = j + b
                nb = 1 - b

                @pl.when(kk + 1 < nj)
                def _():
                    issue(kk + 1, nb)

                @pl.when(kk < nj)
                def _():
                    pltpu.make_async_copy(gsrc_hbm.at[idx_s.at[pl.ds(0, GW)]],
                                          acc.at[b], sem_s[b]).wait()
                    pltpu.make_async_copy(gdst_hbm.at[idx_d.at[pl.ds(0, GW)]],
                                          tmp.at[b], sem_d[b]).wait()

                    @pl.loop(0, GW)
                    def _(r):
                        for c in range(8):
                            sl = pl.ds(c * 16, 16)
                            acc[b, r, sl] = acc[b, r, sl] + tmp[b, r, sl]

                    pltpu.async_copy(
                        acc.at[b], o_hbm.at[pl.ds(row0 + kk * GW, GW)],
                        sem_o[b])

        @pl.when(nj >= 2)
        def _():
            pltpu.make_async_copy(acc.at[0], o_hbm.at[pl.ds(row0, GW)],
                                  so0).wait()
            pltpu.make_async_copy(acc.at[1], o_hbm.at[pl.ds(row0, GW)],
                                  so1).wait()

    pad = 32 * span - n
    if pad:
        src_idx = jnp.pad(src_idx, ((0, 0), (0, pad)))
        dst_idx = jnp.pad(dst_idx, ((0, 0), (0, pad)))
    return k(gsrc, gdst, src_idx, dst_idx)


def _sc_segment_sum(new_edge, dst_idx, zeros):
    n = dst_idx.shape[1]

    @pl.kernel(out_type=jax.ShapeDtypeStruct((2, NPAD, D), F32),
               mesh=_SC_MESH,
               scratch_types=[pltpu.VMEM_SHARED((NPAD, D), F32)])
    def k(ne_hbm, di_hbm, z_hbm, o_hbm, acc):
        cid = lax.axis_index("core")
        sid = lax.axis_index("subcore")
        rows = pl.ds(sid * ROWS_PER_SUB, ROWS_PER_SUB)
        pltpu.sync_copy(z_hbm.at[rows], acc.at[rows])
        plsc.subcore_barrier()

        def body(di_vmem, ne_vmem):
            pltpu.sync_copy(ne_vmem, acc.at[di_vmem.at[0]], add=True)

        pltpu.emit_pipeline(
            body,
            grid=(n // GW,),
            in_specs=[pl.BlockSpec((1, GW), lambda i: (0, i)),
                      pl.BlockSpec((GW, D), lambda i: (i, 0))],
            out_specs=[],
            core_axis_name=("core", "subcore"),
            dimension_semantics=(pltpu.PARALLEL,),
        )(di_hbm, ne_hbm)
        plsc.subcore_barrier()
        pltpu.sync_copy(acc.at[rows], o_hbm.at[cid, rows])

    return k(new_edge, dst_idx, zeros)


def kernel(node_features, edge_features, context_features, edge_index,
           params):
    p = params
    src = [edge_index[0, :NE_H].reshape(1, NE_H),
           edge_index[0, NE_H:].reshape(1, NE_H)]
    dst = [edge_index[1, :NE_H].reshape(1, NE_H),
           edge_index[1, NE_H:].reshape(1, NE_H)]
    zeros = jnp.zeros((NPAD, D), F32)

    w1e = [p["proc"][s]["edge"]["layers"][0][0] for s in range(2)]
    b1e = [p["proc"][s]["edge"]["layers"][0][1] for s in range(2)]
    w1n = [p["proc"][s]["node"]["layers"][0][0] for s in range(2)]
    b1n = [p["proc"][s]["node"]["layers"][0][1] for s in range(2)]

    ctx_lat = _mlp_rows(context_features, p["enc_ctx"])
    node_lat, gsrc, gdst, bee, bne = _enc_node_pre(
        node_features, ctx_lat, p["enc_node"], w1e[0], b1e[0],
        w1n[0], b1n[0])

    el, p2 = [None, None], [None, None]
    g0 = _sc_gather(gsrc, gdst, src[0], dst[0])
    g1 = _sc_gather(gsrc, gdst, src[1], dst[1])
    efp = edge_features.reshape(N_EDGES // 8, 128)
    ebd = jax.scipy.linalg.block_diag(
        *([p["enc_edge"]["layers"][0][0]] * 8))
    for h in range(2):
        ne, el[h] = _edge_step0(h, efp, ebd, (g0, g1)[h],
                                p["enc_edge"], w1e[0], bee,
                                p["proc"][0]["edge"])
        p2[h] = _sc_segment_sum(ne, dst[h], zeros)
    (node_lat, ctx_lat, gsrc, gdst, bee, bne) = _node_step_pre(
        node_lat, p2[0], p2[1], ctx_lat, w1n[0], bne,
        p["proc"][0]["node"], p["proc"][0]["ctx"],
        w1e[1], b1e[1], w1n[1], b1n[1])

    edge_out = None
    g0 = _sc_gather(gsrc, gdst, src[0], dst[0])
    g1 = _sc_gather(gsrc, gdst, src[1], dst[1])
    for h in range(2):
        ne, edge_out = _edge_step1(h, el[h], (g0, g1)[h], w1e[1], bee,
                                   p["proc"][1]["edge"], p["dec_edge"],
                                   eo_prev=edge_out)
        p2[h] = _sc_segment_sum(ne, dst[h], zeros)
    node_out, ctx_out = _node_step_dec(
        node_lat, p2[0], p2[1], ctx_lat, w1n[1], bne,
        p["proc"][1]["node"], p["proc"][1]["ctx"],
        p["dec_node"], p["dec_ctx"])

    return (node_out, edge_out, ctx_out)
